# baseline (speedup 1.0000x reference)
import functools

import jax
import jax.numpy as jnp
from jax import lax
from jax.experimental import pallas as pl
from jax.experimental.pallas import tpu as pltpu
from jax.experimental.pallas import tpu_sc as plsc

_NUM_WORKERS = 32
_GCH = 128
_NB = 5


def _field_lanes(f):
  iota = lax.iota(jnp.int32, 16)
  pos = iota * 5 + f
  return lax.rem(pos, 16), lax.div(pos, 16)


def _make_lookup(n_rows: int, d_out: int):
  assert n_rows % (_NUM_WORKERS * _GCH * _NB) == 0
  rows_per_worker = n_rows // _NUM_WORKERS
  n_groups = rows_per_worker // (_GCH * _NB)
  mesh = plsc.VectorSubcoreMesh(core_axis_name="c", subcore_axis_name="s")

  @functools.partial(
      pl.kernel,
      out_type=jax.ShapeDtypeStruct((n_rows, d_out), jnp.float32),
      mesh=mesh,
      scratch_types=[
          pltpu.VMEM((rows_per_worker * 5,), jnp.int32),
          [pltpu.VMEM((_GCH,), jnp.int32)] * _NB,
          [pltpu.VMEM((_GCH, d_out), jnp.float32)] * _NB,
          pltpu.SemaphoreType.DMA,
          pltpu.SemaphoreType.DMA,
      ],
  )
  def lookup(te_hbm, tab_hbm, out_hbm, te_v, idx_v, rows_v, gsem, osem):
    nc = mesh.num_cores
    wid = lax.axis_index("s") * nc + lax.axis_index("c")
    base = wid * rows_per_worker

    pltpu.sync_copy(te_hbm.at[pl.ds(base * 5, rows_per_worker * 5)], te_v)

    perms = [_field_lanes(f) for f in (2, 3, 4)]

    dnums = lax.GatherDimensionNumbers(
        offset_dims=(), collapsed_slice_dims=(0,), start_index_map=(0,))

    def permute(v, jv):
      return lax.gather(v, jv[:, None], dimension_numbers=dnums,
                        slice_sizes=(1,),
                        mode=lax.GatherScatterMode.PROMISE_IN_BOUNDS)

    def extract(window, f_id):
      jv, kv = perms[f_id]
      out = permute(window[0], jv)
      for k in range(1, 5):
        out = jnp.where(kv == k, permute(window[k], jv), out)
      return out

    def wait_gather(b):
      pltpu.make_async_copy(tab_hbm.at[idx_v[b]], rows_v[b], gsem).wait()

    def fire_out(b, blk):
      pltpu.async_copy(rows_v[b], out_hbm.at[pl.ds(base + blk * _GCH, _GCH)],
                       osem)

    def wait_out(b):
      pltpu.make_async_copy(rows_v[b], out_hbm.at[pl.ds(0, _GCH)],
                            osem).wait()

    def group_body(j2, carry):
      for b in range(_NB):
        blk = j2 * _NB + b
        @pl.when(j2 > 0)
        def _(b=b):
          wait_out(b)

        def idx_body(jj, carry2, b=b, blk=blk):
          p = (blk * 8 + jj) * 80
          window = [te_v[pl.ds(p + 16 * k, 16)] for k in range(5)]
          f2 = extract(window, 0)
          f3 = extract(window, 1)
          f4 = extract(window, 2)
          day = lax.div(lax.rem(f3, 24) * 60 + lax.rem(f4, 60), 5)
          idx_v[b][pl.ds(jj * 16, 16)] = lax.rem(f2, 7) * 288 + day
          return carry2

        lax.fori_loop(0, _GCH // 16, idx_body, 0)
        pltpu.async_copy(tab_hbm.at[idx_v[b]], rows_v[b], gsem)

        if b > 0:
          wait_gather(b - 1)
          fire_out(b - 1, blk - 1)
        else:
          @pl.when(j2 > 0)
          def _(blk=blk):
            wait_gather(_NB - 1)
            fire_out(_NB - 1, blk - 1)
      return carry

    lax.fori_loop(0, n_groups, group_body, 0)

    wait_gather(_NB - 1)
    fire_out(_NB - 1, n_groups * _NB - 1)
    for b in range(_NB):
      wait_out(b)

  return lookup


def kernel(TE, day_table, week_table):
  Bv, Tv, _ = TE.shape
  n_rows = Bv * Tv
  d = day_table.shape[1]
  fused = jnp.concatenate(
      [jnp.tile(day_table, (7, 1)), jnp.repeat(week_table, 288, axis=0)],
      axis=1,
  )
  te_flat = TE.astype(jnp.int32).reshape(n_rows * 5)
  out = _make_lookup(n_rows, 2 * d)(te_flat, fused)
  return out.reshape(Bv, Tv, 2 * d)


# speedup vs baseline: 1.7048x; 1.7048x over previous
import functools

import jax
import jax.numpy as jnp
from jax import lax
from jax.experimental import pallas as pl
from jax.experimental.pallas import tpu as pltpu
from jax.experimental.pallas import tpu_sc as plsc

_NUM_WORKERS = 32
_GCH = 128
_NB = 5


def _field_lanes(f):
  iota = lax.iota(jnp.int32, 16)
  pos = iota * 5 + f
  return lax.rem(pos, 16), lax.div(pos, 16)


def _make_lookup(n_b: int, n_t: int, d_out: int):
  n_rows = n_b * n_t
  assert n_rows % (_NUM_WORKERS * _GCH * _NB) == 0
  rows_per_worker = n_rows // _NUM_WORKERS
  n_groups = rows_per_worker // (_GCH * _NB)
  mesh = plsc.VectorSubcoreMesh(core_axis_name="c", subcore_axis_name="s")

  @functools.partial(
      pl.kernel,
      out_type=jax.ShapeDtypeStruct((n_rows, d_out), jnp.float32),
      mesh=mesh,
      scratch_types=[
          pltpu.VMEM((rows_per_worker * 5,), jnp.int32),
          [pltpu.VMEM((_GCH,), jnp.int32)] * _NB,
          [pltpu.VMEM((_GCH,), jnp.int32)] * _NB,
          [pltpu.VMEM((_GCH, d_out), jnp.float32)] * _NB,
          pltpu.SemaphoreType.DMA,
          pltpu.SemaphoreType.DMA,
      ],
  )
  def lookup(te_hbm, tab_hbm, out_hbm, te_v, idx_v, oidx_v, rows_v, gsem,
             osem):
    nc = mesh.num_cores
    wid = lax.axis_index("s") * nc + lax.axis_index("c")
    base = wid * rows_per_worker

    pltpu.sync_copy(te_hbm.at[pl.ds(base * 5, rows_per_worker * 5)], te_v)

    perms = [_field_lanes(f) for f in (2, 3, 4)]

    dnums = lax.GatherDimensionNumbers(
        offset_dims=(), collapsed_slice_dims=(0,), start_index_map=(0,))

    def permute(v, jv):
      return lax.gather(v, jv[:, None], dimension_numbers=dnums,
                        slice_sizes=(1,),
                        mode=lax.GatherScatterMode.PROMISE_IN_BOUNDS)

    def extract(window, f_id):
      jv, kv = perms[f_id]
      out = permute(window[0], jv)
      for k in range(1, 5):
        out = jnp.where(kv == k, permute(window[k], jv), out)
      return out

    def wait_gather(b):
      pltpu.make_async_copy(tab_hbm.at[idx_v[b]], rows_v[b], gsem).wait()

    def fire_out(b, blk):
      del blk
      pltpu.async_copy(rows_v[b], out_hbm.at[oidx_v[b]], osem)

    def wait_out(b):
      pltpu.make_async_copy(rows_v[b], out_hbm.at[oidx_v[b]], osem).wait()

    def group_body(j2, carry):
      for b in range(_NB):
        blk = j2 * _NB + b
        @pl.when(j2 > 0)
        def _(b=b):
          wait_out(b)

        def idx_body(jj, carry2, b=b, blk=blk):
          p = (blk * 8 + jj) * 80
          window = [te_v[pl.ds(p + 16 * k, 16)] for k in range(5)]
          f2 = extract(window, 0)
          f3 = extract(window, 1)
          f4 = extract(window, 2)
          day = lax.div(lax.rem(f3, 24) * 60 + lax.rem(f4, 60), 5)
          idx_v[b][pl.ds(jj * 16, 16)] = lax.rem(f2, 7) * 288 + day
          rv = base + (blk * 8 + jj) * 16 + lax.iota(jnp.int32, 16)
          oidx_v[b][pl.ds(jj * 16, 16)] = (
              lax.rem(rv, n_t) * n_b + lax.div(rv, n_t))
          return carry2

        lax.fori_loop(0, _GCH // 16, idx_body, 0)
        pltpu.async_copy(tab_hbm.at[idx_v[b]], rows_v[b], gsem)

        if b > 0:
          wait_gather(b - 1)
          fire_out(b - 1, blk - 1)
        else:
          @pl.when(j2 > 0)
          def _(blk=blk):
            wait_gather(_NB - 1)
            fire_out(_NB - 1, blk - 1)
      return carry

    lax.fori_loop(0, n_groups, group_body, 0)

    wait_gather(_NB - 1)
    fire_out(_NB - 1, n_groups * _NB - 1)
    for b in range(_NB):
      wait_out(b)

  return lookup


def kernel(TE, day_table, week_table):
  Bv, Tv, _ = TE.shape
  n_rows = Bv * Tv
  d = day_table.shape[1]
  fused = jnp.concatenate(
      [jnp.tile(day_table, (7, 1)), jnp.repeat(week_table, 288, axis=0)],
      axis=1,
  )
  te_flat = TE.astype(jnp.int32).reshape(n_rows * 5)
  out = _make_lookup(Bv, Tv, 2 * d)(te_flat, fused)
  return out.reshape(Tv, Bv, 2 * d).transpose(1, 0, 2)


# speedup vs baseline: 3.9425x; 2.3126x over previous
import functools

import jax
import jax.numpy as jnp
from jax import lax
from jax.experimental import pallas as pl
from jax.experimental.pallas import tpu as pltpu
from jax.experimental.pallas import tpu_sc as plsc

_NUM_WORKERS = 32
_GCH = 128
_NB = 5


def _make_lookup(n_b: int, n_t: int, d_out: int):
  assert n_b == _NUM_WORKERS * _GCH
  assert n_t % _NB == 0
  n_groups = n_t // _NB
  n_tr = (n_t + 7) // 8
  mesh = plsc.VectorSubcoreMesh(core_axis_name="c", subcore_axis_name="s")

  @functools.partial(
      pl.kernel,
      out_type=jax.ShapeDtypeStruct((n_t * n_b, d_out), jnp.float32),
      mesh=mesh,
      scratch_types=[
          [pltpu.VMEM((n_tr * 8, _GCH), jnp.int32)] * 3,
          [pltpu.VMEM((_GCH,), jnp.int32)] * _NB,
          [pltpu.VMEM((_GCH, d_out), jnp.float32)] * _NB,
          pltpu.SemaphoreType.DMA,
          pltpu.SemaphoreType.DMA,
          pltpu.SemaphoreType.DMA,
      ],
  )
  def lookup(te_hbm, tab_hbm, out_hbm, f_v, idx_v, rows_v, fsem, gsem, osem):
    nc = mesh.num_cores
    wid = lax.axis_index("s") * nc + lax.axis_index("c")
    b0 = wid * _GCH

    for k, f in enumerate((2, 3, 4)):
      for tr in range(n_tr):
        hi = min(8, n_t - tr * 8)
        pltpu.async_copy(te_hbm.at[f, pl.ds(tr * 8, hi), pl.ds(b0, _GCH)],
                         f_v[k].at[pl.ds(tr * 8, hi), :], fsem)
    for k in range(3):
      for tr in range(n_tr):
        hi = min(8, n_t - tr * 8)
        pltpu.make_async_copy(
            te_hbm.at[2, pl.ds(tr * 8, hi), pl.ds(b0, _GCH)],
            f_v[k].at[pl.ds(tr * 8, hi), :], fsem).wait()

    def wait_gather(b):
      pltpu.make_async_copy(tab_hbm.at[idx_v[b]], rows_v[b], gsem).wait()

    def fire_out(b, t):
      pltpu.async_copy(rows_v[b], out_hbm.at[pl.ds(t * n_b + b0, _GCH)],
                       osem)

    def wait_out(b):
      pltpu.make_async_copy(rows_v[b], out_hbm.at[pl.ds(0, _GCH)],
                            osem).wait()

    def group_body(j2, carry):
      for b in range(_NB):
        t = j2 * _NB + b
        @pl.when(j2 > 0)
        def _(b=b):
          wait_out(b)

        def idx_body(jj, carry2, b=b):
          f2 = f_v[0][t, pl.ds(jj * 16, 16)]
          f3 = f_v[1][t, pl.ds(jj * 16, 16)]
          f4 = f_v[2][t, pl.ds(jj * 16, 16)]
          day = lax.div(lax.rem(f3, 24) * 60 + lax.rem(f4, 60), 5)
          idx_v[b][pl.ds(jj * 16, 16)] = lax.rem(f2, 7) * 288 + day
          return carry2

        lax.fori_loop(0, _GCH // 16, idx_body, 0)
        pltpu.async_copy(tab_hbm.at[idx_v[b]], rows_v[b], gsem)

        if b > 0:
          wait_gather(b - 1)
          fire_out(b - 1, t - 1)
        else:
          @pl.when(j2 > 0)
          def _(t=t):
            wait_gather(_NB - 1)
            fire_out(_NB - 1, t - 1)
      return carry

    lax.fori_loop(0, n_groups, group_body, 0)

    wait_gather(_NB - 1)
    fire_out(_NB - 1, n_t - 1)
    for b in range(_NB):
      wait_out(b)

  return lookup


def kernel(TE, day_table, week_table):
  Bv, Tv, _ = TE.shape
  d = day_table.shape[1]
  fused = jnp.concatenate(
      [jnp.tile(day_table, (7, 1)), jnp.repeat(week_table, 288, axis=0)],
      axis=1,
  )
  te_t = jnp.transpose(TE.astype(jnp.int32), (2, 1, 0))
  out = _make_lookup(Bv, Tv, 2 * d)(te_t, fused)
  return out.reshape(Tv, Bv, 2 * d).transpose(1, 0, 2)


# speedup vs baseline: 6.1621x; 1.5630x over previous
import functools

import jax
import jax.numpy as jnp
from jax import lax
from jax.experimental import pallas as pl
from jax.experimental.pallas import tpu as pltpu
from jax.experimental.pallas import tpu_sc as plsc

_NUM_WORKERS = 32
_GCH = 128
_NB = 5


def _make_lookup(n_b: int, n_t: int, d_out: int):
  assert n_b == _NUM_WORKERS * _GCH
  assert n_t % _NB == 0
  n_groups = n_t // _NB
  n_tr = (n_t + 7) // 8
  mesh = plsc.VectorSubcoreMesh(core_axis_name="c", subcore_axis_name="s")

  @functools.partial(
      pl.kernel,
      out_type=jax.ShapeDtypeStruct((n_t * n_b, d_out), jnp.float32),
      mesh=mesh,
      scratch_types=[
          pltpu.VMEM_SHARED((7 * 288, d_out), jnp.float32),
          [pltpu.VMEM((n_tr * 8, _GCH), jnp.int32)] * 3,
          [pltpu.VMEM((_GCH,), jnp.int32)] * _NB,
          [pltpu.VMEM((_GCH, d_out), jnp.float32)] * _NB,
          pltpu.SemaphoreType.DMA,
          pltpu.SemaphoreType.DMA,
          pltpu.SemaphoreType.DMA,
      ],
  )
  def lookup(te_hbm, tab_hbm, out_hbm, tab_sh, f_v, idx_v, rows_v, fsem,
             gsem, osem):
    nc = mesh.num_cores
    wid = lax.axis_index("s") * nc + lax.axis_index("c")
    b0 = wid * _GCH

    @pl.when(lax.axis_index("s") == 0)
    def _():
      pltpu.sync_copy(tab_hbm, tab_sh)
    plsc.subcore_barrier()

    for k, f in enumerate((2, 3, 4)):
      for tr in range(n_tr):
        hi = min(8, n_t - tr * 8)
        pltpu.async_copy(te_hbm.at[f, pl.ds(tr * 8, hi), pl.ds(b0, _GCH)],
                         f_v[k].at[pl.ds(tr * 8, hi), :], fsem)
    for k in range(3):
      for tr in range(n_tr):
        hi = min(8, n_t - tr * 8)
        pltpu.make_async_copy(
            te_hbm.at[2, pl.ds(tr * 8, hi), pl.ds(b0, _GCH)],
            f_v[k].at[pl.ds(tr * 8, hi), :], fsem).wait()

    def wait_gather(b):
      pltpu.make_async_copy(tab_sh.at[idx_v[b]], rows_v[b], gsem).wait()

    def fire_out(b, t):
      pltpu.async_copy(rows_v[b], out_hbm.at[pl.ds(t * n_b + b0, _GCH)],
                       osem)

    def wait_out(b):
      pltpu.make_async_copy(rows_v[b], out_hbm.at[pl.ds(0, _GCH)],
                            osem).wait()

    def group_body(j2, carry):
      for b in range(_NB):
        t = j2 * _NB + b
        @pl.when(j2 > 0)
        def _(b=b):
          wait_out(b)

        def idx_body(jj, carry2, b=b):
          f2 = f_v[0][t, pl.ds(jj * 16, 16)]
          f3 = f_v[1][t, pl.ds(jj * 16, 16)]
          f4 = f_v[2][t, pl.ds(jj * 16, 16)]
          day = lax.div(lax.rem(f3, 24) * 60 + lax.rem(f4, 60), 5)
          idx_v[b][pl.ds(jj * 16, 16)] = lax.rem(f2, 7) * 288 + day
          return carry2

        lax.fori_loop(0, _GCH // 16, idx_body, 0)
        pltpu.async_copy(tab_sh.at[idx_v[b]], rows_v[b], gsem)

        if b > 0:
          wait_gather(b - 1)
          fire_out(b - 1, t - 1)
        else:
          @pl.when(j2 > 0)
          def _(t=t):
            wait_gather(_NB - 1)
            fire_out(_NB - 1, t - 1)
      return carry

    lax.fori_loop(0, n_groups, group_body, 0)

    wait_gather(_NB - 1)
    fire_out(_NB - 1, n_t - 1)
    for b in range(_NB):
      wait_out(b)

  return lookup


def kernel(TE, day_table, week_table):
  Bv, Tv, _ = TE.shape
  d = day_table.shape[1]
  fused = jnp.concatenate(
      [jnp.tile(day_table, (7, 1)), jnp.repeat(week_table, 288, axis=0)],
      axis=1,
  )
  te_t = jnp.transpose(TE.astype(jnp.int32), (2, 1, 0))
  out = _make_lookup(Bv, Tv, 2 * d)(te_t, fused)
  return out.reshape(Tv, Bv, 2 * d).transpose(1, 0, 2)
